# baseline (speedup 1.0000x reference)
import functools

import jax
import jax.numpy as jnp
from jax import lax
from jax.experimental import pallas as pl
from jax.experimental.pallas import tpu as pltpu
from jax.experimental.pallas import tpu_sc as plsc

B, S, H, E, K = 4, 8192, 768, 8, 2
T = B * S
NW = 32
TPW = T // NW
L = 16
GROUPS = TPW // L
CHUNK = 256
NCHK = TPW // CHUNK
GPC = CHUNK // L

TBLK = 1024
NCH = 1
CT = T // NCH
TPC = CT // NW
GPCH = TPC // L


def _tc_dot_body(x_ref, w_ref, o_ref):
    o_ref[...] = jnp.dot(x_ref[...], w_ref[...],
                         preferred_element_type=jnp.float32)


def _tc_dot_chunk(x2d, w128, cid):
    nblk = CT // TBLK
    return pl.pallas_call(
        _tc_dot_body,
        grid=(nblk,),
        in_specs=[
            pl.BlockSpec((TBLK, H), lambda i, c=cid: (c * nblk + i, 0)),
            pl.BlockSpec((H, 128), lambda i: (0, 0)),
        ],
        out_specs=pl.BlockSpec((TBLK, 128), lambda i: (i, 0)),
        out_shape=jax.ShapeDtypeStruct((CT, 128), jnp.float32),
    )(x2d, w128)


def _sc_router_body(dots_hbm, out_hbm, in_v, out_v):
    c = lax.axis_index("c")
    s = lax.axis_index("s")
    wid = s * 2 + c
    base = wid * TPC

    lane = lax.iota(jnp.int32, 16)
    neg_inf = jnp.full((16,), -jnp.inf, jnp.float32)
    one = jnp.full((16,), 1.0, jnp.float32)
    zero = jnp.zeros((16,), jnp.float32)
    ecol = [jnp.full((16,), e, jnp.int32) for e in range(E)]

    pltpu.sync_copy(dots_hbm.at[pl.ds(base, TPC), pl.ds(0, 16)], in_v)

    def zrow(g, _):
        def z16(k, _):
            plsc.store_scatter(out_v,
                               [jnp.full((16,), 0, jnp.int32) + g,
                                k * L + lane], zero)
            return 0
        lax.fori_loop(0, 128 // L, z16, 0)
        return 0
    lax.fori_loop(0, GPCH, zrow, 0)

    def group(g, _):
        rows = g * L + lane
        v = [plsc.load_gather(in_v, [rows, ecol[e]]) for e in range(E)]
        m1 = v[0]
        i1 = jnp.zeros((16,), jnp.int32)
        for e in range(1, E):
            gt = v[e] > m1
            m1 = jnp.where(gt, v[e], m1)
            i1 = jnp.where(gt, e, i1)
        m2 = neg_inf
        i2 = jnp.zeros((16,), jnp.int32)
        for e in range(E):
            cand = jnp.where(i1 == e, neg_inf, v[e])
            gt = cand > m2
            m2 = jnp.where(gt, cand, m2)
            i2 = jnp.where(gt, e, i2)
        e2 = jnp.exp(m2 - m1)
        r1 = one / (one + e2)
        r2 = one - r1
        grow = jnp.full((16,), 0, jnp.int32) + g
        lE = lane * E
        plsc.store_scatter(out_v, [grow, lE + i1], r1)
        plsc.store_scatter(out_v, [grow, lE + i2], r2)
        return 0

    lax.fori_loop(0, GPCH, group, 0)
    pltpu.sync_copy(out_v, out_hbm.at[pl.ds(wid * GPCH, GPCH)])


@functools.partial(
    pl.kernel,
    mesh=plsc.VectorSubcoreMesh(core_axis_name="c", subcore_axis_name="s"),
    out_type=jax.ShapeDtypeStruct((CT * E // 128, 128), jnp.float32),
    scratch_types=[
        pltpu.VMEM((TPC, 16), jnp.float32),
        pltpu.VMEM((GPCH, 128), jnp.float32),
    ],
    compiler_params=pltpu.CompilerParams(needs_layout_passes=False,
                                         use_tc_tiling_on_sc=False),
)
def _sc_router(dots_hbm, out_hbm, in_v, out_v):
    _sc_router_body(dots_hbm, out_hbm, in_v, out_v)


def kernel(x, expert_embeddings):
    x2d = x.reshape(T, H)
    w128 = jnp.zeros((H, 128), jnp.float32).at[:, :E].set(expert_embeddings.T)
    probs = [_sc_router(_tc_dot_chunk(x2d, w128, cid)) for cid in range(NCH)]
    probs = probs[0] if NCH == 1 else jnp.concatenate(probs, axis=0)
    return probs.reshape(B, S, E)


# speedup vs baseline: 1.0694x; 1.0694x over previous
import functools

import jax
import jax.numpy as jnp
from jax import lax
from jax.experimental import pallas as pl
from jax.experimental.pallas import tpu as pltpu
from jax.experimental.pallas import tpu_sc as plsc

B, S, H, E, K = 4, 8192, 768, 8, 2
T = B * S
NSC = 1
NW = NSC * 16
TPW = T // NW
L = 16
GROUPS = TPW // L
CHUNK = 256
NCHK = TPW // CHUNK
GPC = CHUNK // L

TBLK = 4096
NCH = 1
CT = T // NCH
TPC = CT // NW
GPCH = TPC // L


def _tc_dot_body(x_ref, w_ref, o_ref):
    o_ref[...] = jnp.dot(x_ref[...], w_ref[...],
                         preferred_element_type=jnp.float32)


def _tc_dot_chunk(x2d, w128, cid):
    nblk = CT // TBLK
    return pl.pallas_call(
        _tc_dot_body,
        grid=(nblk,),
        in_specs=[
            pl.BlockSpec((TBLK, H), lambda i, c=cid: (c * nblk + i, 0)),
            pl.BlockSpec((H, 128), lambda i: (0, 0)),
        ],
        out_specs=pl.BlockSpec((TBLK, 128), lambda i: (i, 0)),
        out_shape=jax.ShapeDtypeStruct((CT, 128), jnp.float32),
    )(x2d, w128)


def _sc_router_body(dots_hbm, out_hbm, in_v, out_v):
    c = lax.axis_index("c")
    s = lax.axis_index("s")
    wid = s * NSC + c
    base = wid * TPC

    lane = lax.iota(jnp.int32, 16)
    neg_inf = jnp.full((16,), -jnp.inf, jnp.float32)
    one = jnp.full((16,), 1.0, jnp.float32)
    zero = jnp.zeros((16,), jnp.float32)
    ecol = [jnp.full((16,), e, jnp.int32) for e in range(E)]

    pltpu.sync_copy(dots_hbm.at[pl.ds(base, TPC), pl.ds(0, 16)], in_v)

    def zrow(g, _):
        def z16(k, _):
            plsc.store_scatter(out_v,
                               [jnp.full((16,), 0, jnp.int32) + g,
                                k * L + lane], zero)
            return 0
        lax.fori_loop(0, 128 // L, z16, 0)
        return 0
    lax.fori_loop(0, GPCH, zrow, 0)

    def group(g, _):
        rows = g * L + lane
        v = [plsc.load_gather(in_v, [rows, ecol[e]]) for e in range(E)]
        m1 = v[0]
        i1 = jnp.zeros((16,), jnp.int32)
        for e in range(1, E):
            gt = v[e] > m1
            m1 = jnp.where(gt, v[e], m1)
            i1 = jnp.where(gt, e, i1)
        m2 = neg_inf
        i2 = jnp.zeros((16,), jnp.int32)
        for e in range(E):
            cand = jnp.where(i1 == e, neg_inf, v[e])
            gt = cand > m2
            m2 = jnp.where(gt, cand, m2)
            i2 = jnp.where(gt, e, i2)
        e2 = jnp.exp(m2 - m1)
        r1 = one / (one + e2)
        r2 = one - r1
        grow = jnp.full((16,), 0, jnp.int32) + g
        lE = lane * E
        plsc.store_scatter(out_v, [grow, lE + i1], r1)
        plsc.store_scatter(out_v, [grow, lE + i2], r2)
        return 0

    lax.fori_loop(0, GPCH, group, 0)
    pltpu.sync_copy(out_v, out_hbm.at[pl.ds(wid * GPCH, GPCH)])


@functools.partial(
    pl.kernel,
    mesh=plsc.VectorSubcoreMesh(core_axis_name="c", subcore_axis_name="s",
                                num_cores=NSC),
    out_type=jax.ShapeDtypeStruct((CT * E // 128, 128), jnp.float32),
    scratch_types=[
        pltpu.VMEM((TPC, 16), jnp.float32),
        pltpu.VMEM((GPCH, 128), jnp.float32),
    ],
    compiler_params=pltpu.CompilerParams(needs_layout_passes=False,
                                         use_tc_tiling_on_sc=False),
)
def _sc_router(dots_hbm, out_hbm, in_v, out_v):
    _sc_router_body(dots_hbm, out_hbm, in_v, out_v)


def kernel(x, expert_embeddings):
    x2d = x.reshape(T, H)
    w128 = jnp.zeros((H, 128), jnp.float32).at[:, :E].set(expert_embeddings.T)
    probs = [_sc_router(_tc_dot_chunk(x2d, w128, cid)) for cid in range(NCH)]
    probs = probs[0] if NCH == 1 else jnp.concatenate(probs, axis=0)
    return probs.reshape(B, S, E)


# speedup vs baseline: 1.1109x; 1.0389x over previous
import functools

import jax
import jax.numpy as jnp
from jax import lax
from jax.experimental import pallas as pl
from jax.experimental.pallas import tpu as pltpu
from jax.experimental.pallas import tpu_sc as plsc

B, S, H, E, K = 4, 8192, 768, 8, 2
T = B * S
NSC = 2
NW = NSC * 16
TPW = T // NW
L = 16
GROUPS = TPW // L
CHUNK = 256
NCHK = TPW // CHUNK
GPC = CHUNK // L

TBLK = 4096
NCH = 1
CT = T // NCH
TPC = CT // NW
GPCH = TPC // L


def _tc_dot_body(x_ref, w_ref, o_ref):
    o_ref[...] = jnp.dot(x_ref[...], w_ref[...],
                         preferred_element_type=jnp.float32)


def _tc_dot_chunk(x2d, w128, cid):
    nblk = CT // TBLK
    return pl.pallas_call(
        _tc_dot_body,
        grid=(nblk,),
        in_specs=[
            pl.BlockSpec((TBLK, H), lambda i, c=cid: (c * nblk + i, 0)),
            pl.BlockSpec((H, 128), lambda i: (0, 0)),
        ],
        out_specs=pl.BlockSpec((TBLK, 128), lambda i: (i, 0)),
        out_shape=jax.ShapeDtypeStruct((CT, 128), jnp.float32),
    )(x2d, w128)


def _sc_router_body(dots_hbm, out_hbm, in_v, out_v):
    c = lax.axis_index("c")
    s = lax.axis_index("s")
    wid = s * NSC + c
    base = wid * TPC

    lane = lax.iota(jnp.int32, 16)
    neg_inf = jnp.full((16,), -jnp.inf, jnp.float32)
    one = jnp.full((16,), 1.0, jnp.float32)
    zero = jnp.zeros((16,), jnp.float32)
    ecol = [jnp.full((16,), e, jnp.int32) for e in range(E)]

    pltpu.sync_copy(dots_hbm.at[pl.ds(base, TPC), pl.ds(0, 16)], in_v)

    def zrow(g, _):
        def z16(k, _):
            plsc.store_scatter(out_v,
                               [jnp.full((16,), 0, jnp.int32) + g,
                                k * L + lane], zero)
            return 0
        lax.fori_loop(0, 128 // L, z16, 0)
        return 0
    lax.fori_loop(0, GPCH, zrow, 0)

    def group(g, _):
        rows = g * L + lane
        v = [plsc.load_gather(in_v, [rows, ecol[e]]) for e in range(E)]
        m1 = v[0]
        i1 = jnp.zeros((16,), jnp.int32)
        for e in range(1, E):
            gt = v[e] > m1
            m1 = jnp.where(gt, v[e], m1)
            i1 = jnp.where(gt, e, i1)
        m2 = neg_inf
        i2 = jnp.zeros((16,), jnp.int32)
        for e in range(E):
            cand = jnp.where(i1 == e, neg_inf, v[e])
            gt = cand > m2
            m2 = jnp.where(gt, cand, m2)
            i2 = jnp.where(gt, e, i2)
        e2 = jnp.exp(m2 - m1)
        r1 = one / (one + e2)
        r2 = one - r1
        grow = jnp.full((16,), 0, jnp.int32) + g
        lE = lane * E
        plsc.store_scatter(out_v, [grow, lE + i1], r1)
        plsc.store_scatter(out_v, [grow, lE + i2], r2)
        return 0

    lax.fori_loop(0, GPCH, group, 0)
    pltpu.sync_copy(out_v, out_hbm.at[pl.ds(wid * GPCH, GPCH)])


@functools.partial(
    pl.kernel,
    mesh=plsc.VectorSubcoreMesh(core_axis_name="c", subcore_axis_name="s",
                                num_cores=NSC),
    out_type=jax.ShapeDtypeStruct((CT * E // 128, 128), jnp.float32),
    scratch_types=[
        pltpu.VMEM((TPC, 16), jnp.float32),
        pltpu.VMEM((GPCH, 128), jnp.float32),
    ],
    compiler_params=pltpu.CompilerParams(needs_layout_passes=False,
                                         use_tc_tiling_on_sc=False),
)
def _sc_router(dots_hbm, out_hbm, in_v, out_v):
    _sc_router_body(dots_hbm, out_hbm, in_v, out_v)


def kernel(x, expert_embeddings):
    x2d = x.reshape(T, H)
    w128 = jnp.zeros((H, 128), jnp.float32).at[:, :E].set(expert_embeddings.T)
    probs = [_sc_router(_tc_dot_chunk(x2d, w128, cid)) for cid in range(NCH)]
    probs = probs[0] if NCH == 1 else jnp.concatenate(probs, axis=0)
    return probs.reshape(B, S, E)


# speedup vs baseline: 1.1158x; 1.0044x over previous
import functools

import jax
import jax.numpy as jnp
from jax import lax
from jax.experimental import pallas as pl
from jax.experimental.pallas import tpu as pltpu
from jax.experimental.pallas import tpu_sc as plsc

B, S, H, E, K = 4, 8192, 768, 8, 2
T = B * S
NSC = 2
NW = NSC * 16
TPW = T // NW
L = 16
GROUPS = TPW // L
CHUNK = 256
NCHK = TPW // CHUNK
GPC = CHUNK // L

TBLK = 4096
NCH = 1
CT = T // NCH
TPC = CT // NW
GPCH = TPC // L


def _tc_dot_body(x_ref, w_ref, o_ref):
    o_ref[...] = jnp.dot(x_ref[...], w_ref[...],
                         preferred_element_type=jnp.float32)


def _tc_dot_chunk(x2d, w128, cid):
    nblk = CT // TBLK
    return pl.pallas_call(
        _tc_dot_body,
        grid=(nblk,),
        in_specs=[
            pl.BlockSpec((TBLK, H), lambda i, c=cid: (c * nblk + i, 0)),
            pl.BlockSpec((H, 128), lambda i: (0, 0)),
        ],
        out_specs=pl.BlockSpec((TBLK, 128), lambda i: (i, 0)),
        out_shape=jax.ShapeDtypeStruct((CT, 128), jnp.float32),
    )(x2d, w128)


def _sc_router_body(dots_hbm, out_hbm, in_v, out_v, sem):
    c = lax.axis_index("c")
    s = lax.axis_index("s")
    wid = s * NSC + c
    base = wid * TPC

    lane = lax.iota(jnp.int32, 16)
    neg_inf = jnp.full((16,), -jnp.inf, jnp.float32)
    one = jnp.full((16,), 1.0, jnp.float32)
    zero = jnp.zeros((16,), jnp.float32)
    ecol = [jnp.full((16,), e, jnp.int32) for e in range(E)]

    cp = pltpu.async_copy(dots_hbm.at[pl.ds(base, TPC), pl.ds(0, 16)],
                          in_v, sem)

    def zrow(g, _):
        def z16(k, _):
            out_v[g, pl.ds(k * L, L)] = zero
            return 0
        lax.fori_loop(0, 128 // L, z16, 0)
        return 0
    lax.fori_loop(0, GPCH, zrow, 0)
    cp.wait()

    def group(g, _):
        rows = g * L + lane
        v = [plsc.load_gather(in_v, [rows, ecol[e]]) for e in range(E)]
        m1 = v[0]
        i1 = jnp.zeros((16,), jnp.int32)
        for e in range(1, E):
            gt = v[e] > m1
            m1 = jnp.where(gt, v[e], m1)
            i1 = jnp.where(gt, e, i1)
        m2 = neg_inf
        i2 = jnp.zeros((16,), jnp.int32)
        for e in range(E):
            cand = jnp.where(i1 == e, neg_inf, v[e])
            gt = cand > m2
            m2 = jnp.where(gt, cand, m2)
            i2 = jnp.where(gt, e, i2)
        e2 = jnp.exp(m2 - m1)
        r1 = one / (one + e2)
        r2 = one - r1
        grow = jnp.full((16,), 0, jnp.int32) + g
        lE = lane * E
        plsc.store_scatter(out_v, [grow, lE + i1], r1)
        plsc.store_scatter(out_v, [grow, lE + i2], r2)
        return 0

    lax.fori_loop(0, GPCH, group, 0)
    pltpu.sync_copy(out_v, out_hbm.at[pl.ds(wid * GPCH, GPCH)])


@functools.partial(
    pl.kernel,
    mesh=plsc.VectorSubcoreMesh(core_axis_name="c", subcore_axis_name="s",
                                num_cores=NSC),
    out_type=jax.ShapeDtypeStruct((CT * E // 128, 128), jnp.float32),
    scratch_types=[
        pltpu.VMEM((TPC, 16), jnp.float32),
        pltpu.VMEM((GPCH, 128), jnp.float32),
        pltpu.SemaphoreType.DMA,
    ],
    compiler_params=pltpu.CompilerParams(needs_layout_passes=False,
                                         use_tc_tiling_on_sc=False),
)
def _sc_router(dots_hbm, out_hbm, in_v, out_v, sem):
    _sc_router_body(dots_hbm, out_hbm, in_v, out_v, sem)


def kernel(x, expert_embeddings):
    x2d = x.reshape(T, H)
    w128 = jnp.zeros((H, 128), jnp.float32).at[:, :E].set(expert_embeddings.T)
    probs = [_sc_router(_tc_dot_chunk(x2d, w128, cid)) for cid in range(NCH)]
    probs = probs[0] if NCH == 1 else jnp.concatenate(probs, axis=0)
    return probs.reshape(B, S, E)


# speedup vs baseline: 1.1188x; 1.0027x over previous
import functools

import jax
import jax.numpy as jnp
from jax import lax
from jax.experimental import pallas as pl
from jax.experimental.pallas import tpu as pltpu
from jax.experimental.pallas import tpu_sc as plsc

B, S, H, E, K = 4, 8192, 768, 8, 2
T = B * S
NSC = 2
NW = NSC * 16
L = 16

TBLK = 4096
NCH = 1
CT = T // NCH
TPC = CT // NW
GPCH = TPC // L


def _tc_dot_body(x_ref, w_ref, o_ref):
    o_ref[...] = jnp.dot(x_ref[...], w_ref[...],
                         preferred_element_type=jnp.float32)


def _tc_dot_chunk(x2d, w128, cid):
    nblk = CT // TBLK
    return pl.pallas_call(
        _tc_dot_body,
        grid=(nblk,),
        in_specs=[
            pl.BlockSpec((TBLK, H), lambda i, c=cid: (c * nblk + i, 0)),
            pl.BlockSpec((H, 128), lambda i: (0, 0)),
        ],
        out_specs=pl.BlockSpec((TBLK, 128), lambda i: (i, 0)),
        out_shape=jax.ShapeDtypeStruct((CT, 128), jnp.float32),
    )(x2d, w128)


def _sc_router_body(dots_hbm, out_hbm, in_v, out_v, sem):
    c = lax.axis_index("c")
    s = lax.axis_index("s")
    wid = s * NSC + c
    base = wid * TPC

    lane = lax.iota(jnp.int32, 16)
    neg_inf = jnp.full((16,), -jnp.inf, jnp.float32)
    one = jnp.full((16,), 1.0, jnp.float32)
    zero = jnp.zeros((16,), jnp.float32)
    ecol = [jnp.full((16,), e, jnp.int32) for e in range(E)]

    cp = pltpu.async_copy(dots_hbm.at[pl.ds(base, TPC), pl.ds(0, 16)],
                          in_v, sem)

    def zrow(g, _):
        def z16(k, _):
            out_v[g, pl.ds(k * L, L)] = zero
            return 0
        lax.fori_loop(0, 128 // L, z16, 0)
        return 0
    lax.fori_loop(0, GPCH, zrow, 0)
    cp.wait()

    def group(g, _):
        rows = g * L + lane
        v = [plsc.load_gather(in_v, [rows, ecol[e]]) for e in range(E)]
        m1 = v[0]
        i1 = jnp.zeros((16,), jnp.int32)
        for e in range(1, E):
            gt = v[e] > m1
            m1 = jnp.where(gt, v[e], m1)
            i1 = jnp.where(gt, e, i1)
        m2 = neg_inf
        i2 = jnp.zeros((16,), jnp.int32)
        for e in range(E):
            cand = jnp.where(i1 == e, neg_inf, v[e])
            gt = cand > m2
            m2 = jnp.where(gt, cand, m2)
            i2 = jnp.where(gt, e, i2)
        e2 = jnp.exp(m2 - m1)
        r1 = one / (one + e2)
        r2 = one - r1
        grow = jnp.full((16,), 0, jnp.int32) + g
        lE = lane * E
        plsc.store_scatter(out_v, [grow, lE + i1], r1)
        plsc.store_scatter(out_v, [grow, lE + i2], r2)
        return 0

    lax.fori_loop(0, GPCH, group, 0)
    pltpu.sync_copy(out_v, out_hbm.at[pl.ds(wid * GPCH, GPCH)])


@functools.partial(
    pl.kernel,
    mesh=plsc.VectorSubcoreMesh(core_axis_name="c", subcore_axis_name="s",
                                num_cores=NSC),
    out_type=jax.ShapeDtypeStruct((CT * E // 128, 128), jnp.float32),
    scratch_types=[
        pltpu.VMEM((TPC, 16), jnp.float32),
        pltpu.VMEM((GPCH, 128), jnp.float32),
        pltpu.SemaphoreType.DMA,
    ],
    compiler_params=pltpu.CompilerParams(needs_layout_passes=False,
                                         use_tc_tiling_on_sc=False),
)
def _sc_router(dots_hbm, out_hbm, in_v, out_v, sem):
    _sc_router_body(dots_hbm, out_hbm, in_v, out_v, sem)


def kernel(x, expert_embeddings):
    x2d = x.reshape(T, H)
    w128 = jnp.zeros((H, 128), jnp.float32).at[:, :E].set(expert_embeddings.T)
    probs = [_sc_router(_tc_dot_chunk(x2d, w128, cid)) for cid in range(NCH)]
    probs = probs[0] if NCH == 1 else jnp.concatenate(probs, axis=0)
    return probs.reshape(B, S, E)
